# baseline (speedup 1.0000x reference)
import functools

import jax
import jax.numpy as jnp
from jax import lax
from jax.experimental import pallas as pl
from jax.experimental.pallas import tpu as pltpu
from jax.experimental.pallas import tpu_sc as plsc

B = 16384
L = 200
H = 256
V = 1000
N_TOKENS = B * L

_NC = 2
_NS = 16
_NW = _NC * _NS

CHUNK = 128
G = 16
SB = 2 * G
TOK_PER_W = N_TOKENS // _NW
CHUNKS_PER_W = TOK_PER_W // CHUNK
T_OUT = CHUNKS_PER_W // SB
_LANES = 16
_IDX_VREGS = CHUNK // _LANES

assert TOK_PER_W % L == 0
assert CHUNKS_PER_W % SB == 0


def _fuse_body(word_ref, pos_ref, out_ref):
    out_ref[...] = word_ref[...][None, :, :] + pos_ref[...][:, None, :]


def _build_fused(word_ebd, pos_ebd):
    lb = 8
    fused = pl.pallas_call(
        _fuse_body,
        grid=(L // lb,),
        in_specs=[
            pl.BlockSpec((V, H), lambda l: (0, 0)),
            pl.BlockSpec((lb, H), lambda l: (l, 0)),
        ],
        out_specs=pl.BlockSpec((lb, V, H), lambda l: (l, 0, 0)),
        out_shape=jax.ShapeDtypeStruct((L, V, H), jnp.float32),
    )(word_ebd, pos_ebd)
    return fused.reshape(L * V, H)


def _to_fused_rows(idx_ref, off, p, lane):
    for k in range(_IDX_VREGS):
        sl = pl.ds(off + k * _LANES, _LANES)
        t = lane + (p + k * _LANES)
        lmod = t - jnp.where(t >= L, L, 0)
        idx_ref[sl] = idx_ref[sl] + lmod * V


def _ebd_kernel(x_hbm, fused_hbm, out_hbm,
                ibuf0, ibuf1, rows0, rows1,
                isem0, isem1, gsem0, gsem1, ssem0, ssem1):
    wid = lax.axis_index("s") * _NC + lax.axis_index("c")
    w_base = wid * TOK_PER_W
    ibuf = (ibuf0, ibuf1)
    isem = (isem0, isem1)
    rows = (rows0, rows1)
    gsem = (gsem0, gsem1)
    ssem = (ssem0, ssem1)
    lane = lax.iota(jnp.int32, _LANES)

    pltpu.sync_copy(x_hbm.at[pl.ds(w_base, G * CHUNK)], ibuf0)
    _to_fused_rows(ibuf0, 0, jnp.int32(0), lane)
    pltpu.async_copy(fused_hbm.at[ibuf0.at[pl.ds(0, CHUNK)]], rows0, gsem0)

    def outer(t, p):
        sb_base = w_base + t * SB * CHUNK
        for j in range(SB):
            c_base = sb_base + j * CHUNK
            b2 = j % 2
            nb = 1 - b2

            if j == 0:
                pltpu.async_copy(
                    x_hbm.at[pl.ds(sb_base + G * CHUNK, G * CHUNK)],
                    ibuf1, isem1)
            if j == G:
                @pl.when(t < T_OUT - 1)
                def _prefetch_next_sb():
                    pltpu.async_copy(
                        x_hbm.at[pl.ds(sb_base + SB * CHUNK, G * CHUNK)],
                        ibuf0, isem0)

            pn = p + (CHUNK % L)
            pn = pn - jnp.where(pn >= L, L, 0)

            nib = ((j + 1) // G) % 2
            noff = ((j + 1) % G) * CHUNK

            def _issue(j=j, nb=nb, nib=nib, noff=noff, pn=pn):
                if j == G - 1 or j == SB - 1:
                    pltpu.make_async_copy(
                        x_hbm.at[pl.ds(0, G * CHUNK)], ibuf[nib], isem[nib]
                    ).wait()
                _to_fused_rows(ibuf[nib], noff, pn, lane)
                if not j == 0:
                    pltpu.make_async_copy(
                        rows[nb], out_hbm.at[pl.ds(0, CHUNK)], ssem[nb]
                    ).wait()
                pltpu.async_copy(
                    fused_hbm.at[ibuf[nib].at[pl.ds(noff, CHUNK)]],
                    rows[nb], gsem[nb])

            if j == 0:
                def _issue0(nb=nb, nib=nib, noff=noff, pn=pn):
                    _to_fused_rows(ibuf[nib], noff, pn, lane)

                    @pl.when(t > 0)
                    def _drain():
                        pltpu.make_async_copy(
                            rows[nb], out_hbm.at[pl.ds(0, CHUNK)], ssem[nb]
                        ).wait()

                    pltpu.async_copy(
                        fused_hbm.at[ibuf[nib].at[pl.ds(noff, CHUNK)]],
                        rows[nb], gsem[nb])

                _issue0()
            elif j == SB - 1:
                pl.when(t < T_OUT - 1)(_issue)
            else:
                _issue()

            pltpu.make_async_copy(
                fused_hbm.at[pl.ds(0, CHUNK)], rows[b2], gsem[b2]).wait()
            pltpu.async_copy(rows[b2], out_hbm.at[pl.ds(c_base, CHUNK)],
                             ssem[b2])
            p = pn
        return p

    lax.fori_loop(0, T_OUT, outer, jnp.int32(0), unroll=False)

    pltpu.make_async_copy(rows0, out_hbm.at[pl.ds(0, CHUNK)], ssem0).wait()
    pltpu.make_async_copy(rows1, out_hbm.at[pl.ds(0, CHUNK)], ssem1).wait()


@jax.jit
def _run(x_flat, word_ebd, pos_ebd):
    fused = _build_fused(word_ebd, pos_ebd)
    mesh = plsc.VectorSubcoreMesh(core_axis_name="c", subcore_axis_name="s")
    f = functools.partial(
        pl.kernel,
        mesh=mesh,
        out_type=jax.ShapeDtypeStruct((N_TOKENS, H), jnp.float32),
        scratch_types=[
            pltpu.VMEM((G * CHUNK,), jnp.int32),
            pltpu.VMEM((G * CHUNK,), jnp.int32),
            pltpu.VMEM((CHUNK, H), jnp.float32),
            pltpu.VMEM((CHUNK, H), jnp.float32),
            pltpu.SemaphoreType.DMA,
            pltpu.SemaphoreType.DMA,
            pltpu.SemaphoreType.DMA,
            pltpu.SemaphoreType.DMA,
            pltpu.SemaphoreType.DMA,
            pltpu.SemaphoreType.DMA,
        ],
    )(_ebd_kernel)
    return f(x_flat, fused)


def kernel(X, word_ebd, pos_ebd):
    x_flat = X.reshape(-1).astype(jnp.int32)
    out = _run(x_flat, word_ebd, pos_ebd)
    return out.reshape(B, L, H)


# speedup vs baseline: 1.0010x; 1.0010x over previous
import functools

import jax
import jax.numpy as jnp
from jax import lax
from jax.experimental import pallas as pl
from jax.experimental.pallas import tpu as pltpu
from jax.experimental.pallas import tpu_sc as plsc

B = 16384
L = 200
H = 256
V = 1000
N_TOKENS = B * L

_NC = 2
_NS = 16
_NW = _NC * _NS

CHUNK = 64
G = 16
SB = 2 * G
NBUF = 4
TOK_PER_W = N_TOKENS // _NW
CHUNKS_PER_W = TOK_PER_W // CHUNK
T_OUT = CHUNKS_PER_W // SB
_LANES = 16
_IDX_VREGS = CHUNK // _LANES
_PSTEP = CHUNK % L

assert TOK_PER_W % L == 0
assert CHUNKS_PER_W % SB == 0 and SB % NBUF == 0 and SB == 2 * G


def _fuse_body(word_ref, pos_ref, out_ref):
    out_ref[...] = word_ref[...][None, :, :] + pos_ref[...][:, None, :]


def _build_fused(word_ebd, pos_ebd):
    lb = 8
    fused = pl.pallas_call(
        _fuse_body,
        grid=(L // lb,),
        in_specs=[
            pl.BlockSpec((V, H), lambda l: (0, 0)),
            pl.BlockSpec((lb, H), lambda l: (l, 0)),
        ],
        out_specs=pl.BlockSpec((lb, V, H), lambda l: (l, 0, 0)),
        out_shape=jax.ShapeDtypeStruct((L, V, H), jnp.float32),
    )(word_ebd, pos_ebd)
    return fused.reshape(L * V, H)


def _to_fused_rows(idx_ref, off, p, lane):
    for k in range(_IDX_VREGS):
        sl = pl.ds(off + k * _LANES, _LANES)
        t = lane + (p + k * _LANES)
        lmod = t - jnp.where(t >= L, L, 0)
        idx_ref[sl] = idx_ref[sl] + lmod * V


def _ebd_kernel(x_hbm, fused_hbm, out_hbm,
                ibuf0, ibuf1, rows0, rows1, rows2, rows3,
                isem0, isem1, gsem0, gsem1, gsem2, gsem3,
                ssem0, ssem1, ssem2, ssem3):
    wid = lax.axis_index("s") * _NC + lax.axis_index("c")
    w_base = wid * TOK_PER_W
    ibuf = (ibuf0, ibuf1)
    isem = (isem0, isem1)
    rows = (rows0, rows1, rows2, rows3)
    gsem = (gsem0, gsem1, gsem2, gsem3)
    ssem = (ssem0, ssem1, ssem2, ssem3)
    lane = lax.iota(jnp.int32, _LANES)

    pltpu.sync_copy(x_hbm.at[pl.ds(w_base, G * CHUNK)], ibuf0)
    _to_fused_rows(ibuf0, 0, jnp.int32(0), lane)
    pltpu.async_copy(fused_hbm.at[ibuf0.at[pl.ds(0, CHUNK)]], rows0, gsem0)
    _to_fused_rows(ibuf0, CHUNK, jnp.int32(_PSTEP), lane)
    pltpu.async_copy(fused_hbm.at[ibuf0.at[pl.ds(CHUNK, CHUNK)]], rows1, gsem1)

    def outer(t, q):
        sb_base = w_base + t * SB * CHUNK
        for j in range(SB):
            c_base = sb_base + j * CHUNK
            rb = j % NBUF
            fb = (j + 2) % NBUF

            if j == 0:
                pltpu.async_copy(
                    x_hbm.at[pl.ds(sb_base + G * CHUNK, G * CHUNK)],
                    ibuf1, isem1)
            if j == G:
                @pl.when(t < T_OUT - 1)
                def _prefetch_next_sb():
                    pltpu.async_copy(
                        x_hbm.at[pl.ds(sb_base + SB * CHUNK, G * CHUNK)],
                        ibuf0, isem0)

            nib = ((j + 2) // G) % 2
            noff = ((j + 2) % G) * CHUNK

            def _issue(j=j, fb=fb, nib=nib, noff=noff, q=q):
                if j == G - 2 or j == SB - 2:
                    pltpu.make_async_copy(
                        x_hbm.at[pl.ds(0, G * CHUNK)], ibuf[nib], isem[nib]
                    ).wait()
                _to_fused_rows(ibuf[nib], noff, q, lane)
                if j in (0, 1):
                    @pl.when(t > 0)
                    def _drain():
                        pltpu.make_async_copy(
                            rows[fb], out_hbm.at[pl.ds(0, CHUNK)], ssem[fb]
                        ).wait()
                else:
                    pltpu.make_async_copy(
                        rows[fb], out_hbm.at[pl.ds(0, CHUNK)], ssem[fb]
                    ).wait()
                pltpu.async_copy(
                    fused_hbm.at[ibuf[nib].at[pl.ds(noff, CHUNK)]],
                    rows[fb], gsem[fb])

            if j >= SB - 2:
                pl.when(t < T_OUT - 1)(_issue)
            else:
                _issue()

            pltpu.make_async_copy(
                fused_hbm.at[pl.ds(0, CHUNK)], rows[rb], gsem[rb]).wait()
            pltpu.async_copy(rows[rb], out_hbm.at[pl.ds(c_base, CHUNK)],
                             ssem[rb])

            q = q + _PSTEP
            q = q - jnp.where(q >= L, L, 0)
        return q

    lax.fori_loop(0, T_OUT, outer, jnp.int32(2 * _PSTEP), unroll=False)

    for r, s in zip(rows, ssem):
        pltpu.make_async_copy(r, out_hbm.at[pl.ds(0, CHUNK)], s).wait()


@jax.jit
def _run(x_flat, word_ebd, pos_ebd):
    fused = _build_fused(word_ebd, pos_ebd)
    mesh = plsc.VectorSubcoreMesh(core_axis_name="c", subcore_axis_name="s")
    f = functools.partial(
        pl.kernel,
        mesh=mesh,
        out_type=jax.ShapeDtypeStruct((N_TOKENS, H), jnp.float32),
        scratch_types=(
            [pltpu.VMEM((G * CHUNK,), jnp.int32)] * 2
            + [pltpu.VMEM((CHUNK, H), jnp.float32)] * NBUF
            + [pltpu.SemaphoreType.DMA] * (2 + 2 * NBUF)
        ),
    )(_ebd_kernel)
    return f(x_flat, fused)


def kernel(X, word_ebd, pos_ebd):
    x_flat = X.reshape(-1).astype(jnp.int32)
    out = _run(x_flat, word_ebd, pos_ebd)
    return out.reshape(B, L, H)
